# baseline (speedup 1.0000x reference)
import functools

import jax
import jax.numpy as jnp
from jax import lax
from jax.experimental import pallas as pl
from jax.experimental.pallas import tpu as pltpu
from jax.experimental.pallas import tpu_sc as plsc

N_TOK = 8192
EMB_DIM = 256
N_EMB = 8192

BM, BN = 1024, 1024
MB_A, NB_A = N_TOK // BM, N_EMB // BN


def _dist_body(x_ref, e_ref, dist_ref, idx_ref, minv, argv, x2c, colsc):
    n = pl.program_id(1)
    mdim = pl.program_id(0)

    @pl.when((mdim == 0) & (n == 0))
    def _():
        colsc[...] = lax.broadcasted_iota(
            jnp.int32, (BM, BN), 1).astype(jnp.float32)

    x = x_ref[...]
    e = e_ref[...]
    m = lax.dot_general(-2.0 * x, e, (((1,), (0,)), ((), ())),
                        preferred_element_type=jnp.float32)

    @pl.when(n == 0)
    def _():
        x2c[...] = jnp.sum(jnp.square(x), axis=1, keepdims=True)

    x2 = x2c[...]
    e2 = jnp.sum(jnp.square(e), axis=0, keepdims=True)
    dist = (x2 + m) + e2
    dist_ref[...] = dist

    lmin = jnp.min(dist, axis=1, keepdims=True)
    larg = jnp.min(jnp.where(dist == lmin, colsc[...], jnp.float32(3.0e38)),
                   axis=1, keepdims=True) + (n * BN)

    @pl.when(n == 0)
    def _():
        minv[...] = lmin
        argv[...] = larg

    @pl.when(n > 0)
    def _():
        better = lmin < minv[...]
        argv[...] = jnp.where(better, larg, argv[...])
        minv[...] = jnp.minimum(lmin, minv[...])

    @pl.when(n == NB_A - 1)
    def _():
        idx_ref[0] = argv[...].astype(jnp.int32)


_dist_call = pl.pallas_call(
    _dist_body,
    grid=(MB_A, NB_A),
    in_specs=[
        pl.BlockSpec((BM, EMB_DIM), lambda m, n: (m, 0)),
        pl.BlockSpec((EMB_DIM, BN), lambda m, n: (0, n)),
    ],
    out_specs=[
        pl.BlockSpec((BM, BN), lambda m, n: (m, n)),
        pl.BlockSpec((1, BM, 1), lambda m, n: (m, 0, 0)),
    ],
    out_shape=[
        jax.ShapeDtypeStruct((N_TOK, N_EMB), jnp.float32),
        jax.ShapeDtypeStruct((MB_A, BM, 1), jnp.int32),
    ],
    scratch_shapes=[
        pltpu.VMEM((BM, 1), jnp.float32),
        pltpu.VMEM((BM, 1), jnp.float32),
        pltpu.VMEM((BM, 1), jnp.float32),
        pltpu.VMEM((BM, BN), jnp.float32),
    ],
    compiler_params=pltpu.CompilerParams(
        dimension_semantics=("arbitrary", "arbitrary")),
)

BM_C, BN_C = 1024, 1024
MB_C, NB_C = N_TOK // BM_C, N_EMB // BN_C


def _onehot_body(idx_ref, enc_ref, perp_ref, colsum, ent):
    nn = pl.program_id(0)
    mm = pl.program_id(1)
    idx = idx_ref[0]
    cols = nn * BN_C + lax.broadcasted_iota(jnp.int32, (BM_C, BN_C), 1)
    oh = (idx == cols).astype(jnp.float32)
    enc_ref[...] = oh
    cs = jnp.sum(oh, axis=0, keepdims=True)

    @pl.when(mm == 0)
    def _():
        colsum[...] = cs

    @pl.when(mm > 0)
    def _():
        colsum[...] = colsum[...] + cs

    @pl.when(mm == MB_C - 1)
    def _():
        p = colsum[...] * (1.0 / N_TOK)
        ent_part = jnp.sum(p * jnp.log(p + 1e-10), axis=1, keepdims=True)
        prev = jnp.where(nn == 0, jnp.zeros_like(ent_part), ent[...])
        ent[...] = prev + ent_part

    @pl.when((mm == MB_C - 1) & (nn == NB_C - 1))
    def _():
        perp_ref[...] = jnp.exp(-ent[...])


_onehot_call = pl.pallas_call(
    _onehot_body,
    grid=(NB_C, MB_C),
    in_specs=[
        pl.BlockSpec((1, BM_C, 1), lambda nn, mm: (mm, 0, 0)),
    ],
    out_specs=[
        pl.BlockSpec((BM_C, BN_C), lambda nn, mm: (mm, nn)),
        pl.BlockSpec((1, 1), lambda nn, mm: (0, 0)),
    ],
    out_shape=[
        jax.ShapeDtypeStruct((N_TOK, N_EMB), jnp.float32),
        jax.ShapeDtypeStruct((1, 1), jnp.float32),
    ],
    scratch_shapes=[
        pltpu.VMEM((1, BN_C), jnp.float32),
        pltpu.VMEM((1, 1), jnp.float32),
    ],
    compiler_params=pltpu.CompilerParams(
        dimension_semantics=("arbitrary", "arbitrary")),
)

BM_D = 1024
MB_D = N_TOK // BM_D


def _loss_body(x_ref, q_ref, loss_ref, acc):
    i = pl.program_id(0)
    d = x_ref[...] - q_ref[...]
    s = jnp.sum(d * d).reshape(1, 1)
    prev = jnp.where(i == 0, jnp.zeros_like(s), acc[...])
    tot = prev + s
    acc[...] = tot

    @pl.when(i == MB_D - 1)
    def _():
        loss_ref[...] = tot * (1.25 / (N_TOK * EMB_DIM))


_loss_call = pl.pallas_call(
    _loss_body,
    grid=(MB_D,),
    in_specs=[
        pl.BlockSpec((BM_D, EMB_DIM), lambda i: (i, 0)),
        pl.BlockSpec((BM_D, EMB_DIM), lambda i: (i, 0)),
    ],
    out_specs=pl.BlockSpec((1, 1), lambda i: (0, 0)),
    out_shape=jax.ShapeDtypeStruct((1, 1), jnp.float32),
    scratch_shapes=[pltpu.VMEM((1, 1), jnp.float32)],
    compiler_params=pltpu.CompilerParams(
        dimension_semantics=("arbitrary",)),
)


@functools.lru_cache(maxsize=1)
def _make_sc_gather():
    info = plsc.get_sparse_core_info()
    nc, ns = info.num_cores, info.num_subcores
    nw = nc * ns
    bpw = N_TOK // nw
    mesh = plsc.VectorSubcoreMesh(core_axis_name="c", subcore_axis_name="s")

    @functools.partial(
        pl.kernel, mesh=mesh,
        out_type=jax.ShapeDtypeStruct((N_TOK, EMB_DIM), jnp.float32),
        scratch_types=[
            pltpu.VMEM((bpw,), jnp.int32),
            pltpu.VMEM((bpw, EMB_DIM), jnp.float32),
            pltpu.SemaphoreType.DMA,
        ],
    )
    def gk(table_hbm, idx_hbm, out_hbm, idx_v, rows_v, sem):
        wid = lax.axis_index("s") * nc + lax.axis_index("c")
        base = wid * bpw
        pltpu.sync_copy(idx_hbm.at[pl.ds(base, bpw)], idx_v)
        pltpu.async_copy(table_hbm.at[idx_v], rows_v, sem).wait()
        pltpu.sync_copy(rows_v, out_hbm.at[pl.ds(base, bpw)])

    return gk


def kernel(inputs, embeddings):
    flat = inputs.reshape(N_TOK, EMB_DIM)
    distances, idx3 = _dist_call(flat, embeddings)
    idx_flat = idx3.reshape(N_TOK)
    encodings, perp = _onehot_call(idx3)
    table = embeddings.T
    quantized = _make_sc_gather()(table, idx_flat)
    loss = _loss_call(flat, quantized)
    quantized_st = quantized.reshape(inputs.shape)
    encoding_indices = idx_flat.reshape(inputs.shape[:-1])
    return (quantized_st, loss.reshape(()), perp.reshape(()),
            encodings, encoding_indices, distances)


# speedup vs baseline: 1.1105x; 1.1105x over previous
import functools

import jax
import jax.numpy as jnp
from jax import lax
from jax.experimental import pallas as pl
from jax.experimental.pallas import tpu as pltpu
from jax.experimental.pallas import tpu_sc as plsc

N_TOK = 8192
EMB_DIM = 256
N_EMB = 8192

BM, BN = 2048, 1024
MB_A, NB_A = N_TOK // BM, N_EMB // BN


def _dist_body(x_ref, e_ref, dist_ref, idx_ref, et_ref, minv, argv):
    mdim = pl.program_id(0)
    n = pl.program_id(1)
    x = x_ref[...]
    e = e_ref[...]

    @pl.when(mdim == 0)
    def _():
        et_ref[pl.ds(n * BN, BN), :] = e.T

    m = lax.dot_general(x, e, (((1,), (0,)), ((), ())),
                        preferred_element_type=jnp.float32)
    x2 = jnp.sum(jnp.square(x), axis=1, keepdims=True)
    e2 = jnp.sum(jnp.square(e), axis=0, keepdims=True)
    dist = (x2 - 2.0 * m) + e2
    dist_ref[...] = dist

    lmin = jnp.min(dist, axis=1, keepdims=True)
    cols = n * BN + lax.broadcasted_iota(jnp.int32, (BM, BN), 1)
    larg = jnp.min(jnp.where(dist == lmin, cols, jnp.int32(2147483647)),
                   axis=1, keepdims=True)

    @pl.when(n == 0)
    def _():
        minv[...] = lmin
        argv[...] = larg

    @pl.when(n > 0)
    def _():
        better = lmin < minv[...]
        argv[...] = jnp.where(better, larg, argv[...])
        minv[...] = jnp.minimum(lmin, minv[...])

    @pl.when(n == NB_A - 1)
    def _():
        idx_ref[0] = argv[...]


_dist_call = pl.pallas_call(
    _dist_body,
    grid=(MB_A, NB_A),
    in_specs=[
        pl.BlockSpec((BM, EMB_DIM), lambda m, n: (m, 0)),
        pl.BlockSpec((EMB_DIM, BN), lambda m, n: (0, n)),
    ],
    out_specs=[
        pl.BlockSpec((BM, BN), lambda m, n: (m, n)),
        pl.BlockSpec((1, BM, 1), lambda m, n: (m, 0, 0)),
        pl.BlockSpec((N_EMB, EMB_DIM), lambda m, n: (0, 0)),
    ],
    out_shape=[
        jax.ShapeDtypeStruct((N_TOK, N_EMB), jnp.float32),
        jax.ShapeDtypeStruct((MB_A, BM, 1), jnp.int32),
        jax.ShapeDtypeStruct((N_EMB, EMB_DIM), jnp.float32),
    ],
    scratch_shapes=[
        pltpu.VMEM((BM, 1), jnp.float32),
        pltpu.VMEM((BM, 1), jnp.int32),
    ],
    compiler_params=pltpu.CompilerParams(
        dimension_semantics=("arbitrary", "arbitrary")),
)

BM_C, BN_C = 1024, 1024
MB_C, NB_C = N_TOK // BM_C, N_EMB // BN_C


def _onehot_body(idx_ref, enc_ref, perp_ref, colsum, ent):
    nn = pl.program_id(0)
    mm = pl.program_id(1)
    idx = idx_ref[0]
    cols = nn * BN_C + lax.broadcasted_iota(jnp.int32, (BM_C, BN_C), 1)
    oh = (idx == cols).astype(jnp.float32)
    enc_ref[...] = oh
    cs = jnp.sum(oh, axis=0, keepdims=True)

    @pl.when(mm == 0)
    def _():
        colsum[...] = cs

    @pl.when(mm > 0)
    def _():
        colsum[...] = colsum[...] + cs

    @pl.when(mm == MB_C - 1)
    def _():
        p = colsum[...] * (1.0 / N_TOK)
        ent_part = jnp.sum(p * jnp.log(p + 1e-10), axis=1, keepdims=True)
        prev = jnp.where(nn == 0, jnp.zeros_like(ent_part), ent[...])
        ent[...] = prev + ent_part

    @pl.when((mm == MB_C - 1) & (nn == NB_C - 1))
    def _():
        perp_ref[...] = jnp.exp(-ent[...])


_onehot_call = pl.pallas_call(
    _onehot_body,
    grid=(NB_C, MB_C),
    in_specs=[
        pl.BlockSpec((1, BM_C, 1), lambda nn, mm: (mm, 0, 0)),
    ],
    out_specs=[
        pl.BlockSpec((BM_C, BN_C), lambda nn, mm: (mm, nn)),
        pl.BlockSpec((1, 1), lambda nn, mm: (0, 0)),
    ],
    out_shape=[
        jax.ShapeDtypeStruct((N_TOK, N_EMB), jnp.float32),
        jax.ShapeDtypeStruct((1, 1), jnp.float32),
    ],
    scratch_shapes=[
        pltpu.VMEM((1, BN_C), jnp.float32),
        pltpu.VMEM((1, 1), jnp.float32),
    ],
    compiler_params=pltpu.CompilerParams(
        dimension_semantics=("arbitrary", "arbitrary")),
)

BM_D = 1024
MB_D = N_TOK // BM_D


def _loss_body(x_ref, q_ref, loss_ref, acc):
    i = pl.program_id(0)
    d = x_ref[...] - q_ref[...]
    s = jnp.sum(d * d).reshape(1, 1)
    prev = jnp.where(i == 0, jnp.zeros_like(s), acc[...])
    tot = prev + s
    acc[...] = tot

    @pl.when(i == MB_D - 1)
    def _():
        loss_ref[...] = tot * (1.25 / (N_TOK * EMB_DIM))


_loss_call = pl.pallas_call(
    _loss_body,
    grid=(MB_D,),
    in_specs=[
        pl.BlockSpec((BM_D, EMB_DIM), lambda i: (i, 0)),
        pl.BlockSpec((BM_D, EMB_DIM), lambda i: (i, 0)),
    ],
    out_specs=pl.BlockSpec((1, 1), lambda i: (0, 0)),
    out_shape=jax.ShapeDtypeStruct((1, 1), jnp.float32),
    scratch_shapes=[pltpu.VMEM((1, 1), jnp.float32)],
    compiler_params=pltpu.CompilerParams(
        dimension_semantics=("arbitrary",)),
)


@functools.lru_cache(maxsize=1)
def _make_sc_gather():
    info = plsc.get_sparse_core_info()
    nc, ns = info.num_cores, info.num_subcores
    nw = nc * ns
    bpw = N_TOK // nw
    mesh = plsc.VectorSubcoreMesh(core_axis_name="c", subcore_axis_name="s")

    @functools.partial(
        pl.kernel, mesh=mesh,
        out_type=jax.ShapeDtypeStruct((N_TOK, EMB_DIM), jnp.float32),
        scratch_types=[
            pltpu.VMEM((bpw,), jnp.int32),
            pltpu.VMEM((bpw, EMB_DIM), jnp.float32),
            pltpu.SemaphoreType.DMA,
        ],
    )
    def gk(table_hbm, idx_hbm, out_hbm, idx_v, rows_v, sem):
        wid = lax.axis_index("s") * nc + lax.axis_index("c")
        base = wid * bpw
        pltpu.sync_copy(idx_hbm.at[pl.ds(base, bpw)], idx_v)
        pltpu.async_copy(table_hbm.at[idx_v], rows_v, sem).wait()
        pltpu.sync_copy(rows_v, out_hbm.at[pl.ds(base, bpw)])

    return gk


def kernel(inputs, embeddings):
    flat = inputs.reshape(N_TOK, EMB_DIM)
    distances, idx3, table = _dist_call(flat, embeddings)
    idx_flat = idx3.reshape(N_TOK)
    encodings, perp = _onehot_call(idx3.reshape(MB_C, BM_C, 1))
    quantized = _make_sc_gather()(table, idx_flat)
    loss = _loss_call(flat, quantized)
    quantized_st = quantized.reshape(inputs.shape)
    encoding_indices = idx_flat.reshape(inputs.shape[:-1])
    return (quantized_st, loss.reshape(()), perp.reshape(()),
            encodings, encoding_indices, distances)


# speedup vs baseline: 1.1903x; 1.0719x over previous
import functools

import jax
import jax.numpy as jnp
from jax import lax
from jax.experimental import pallas as pl
from jax.experimental.pallas import tpu as pltpu
from jax.experimental.pallas import tpu_sc as plsc

N_TOK = 8192
EMB_DIM = 256
N_EMB = 8192

BM, BN = 2048, 1024
MB_A, NB_A = N_TOK // BM, N_EMB // BN


def _dist_body(x_ref, e_ref, dist_ref, idx_ref, et_ref, minv, argv):
    mdim = pl.program_id(0)
    n = pl.program_id(1)
    x = x_ref[...]
    e = e_ref[...]

    @pl.when(mdim == 0)
    def _():
        et_ref[pl.ds(n * BN, BN), :] = e.T

    m = lax.dot_general(x, e, (((1,), (0,)), ((), ())),
                        preferred_element_type=jnp.float32)
    x2 = jnp.sum(jnp.square(x), axis=1, keepdims=True)
    e2 = jnp.sum(jnp.square(e), axis=0, keepdims=True)
    dist = (x2 - 2.0 * m) + e2
    dist_ref[...] = dist

    lmin = jnp.min(dist, axis=1, keepdims=True)
    cols = n * BN + lax.broadcasted_iota(jnp.int32, (BM, BN), 1)
    larg = jnp.min(jnp.where(dist == lmin, cols, jnp.int32(2147483647)),
                   axis=1, keepdims=True)

    @pl.when(n == 0)
    def _():
        minv[...] = lmin
        argv[...] = larg

    @pl.when(n > 0)
    def _():
        better = lmin < minv[...]
        argv[...] = jnp.where(better, larg, argv[...])
        minv[...] = jnp.minimum(lmin, minv[...])

    @pl.when(n == NB_A - 1)
    def _():
        idx_ref[0] = argv[...]


_dist_call = pl.pallas_call(
    _dist_body,
    grid=(MB_A, NB_A),
    in_specs=[
        pl.BlockSpec((BM, EMB_DIM), lambda m, n: (m, 0)),
        pl.BlockSpec((EMB_DIM, BN), lambda m, n: (0, n)),
    ],
    out_specs=[
        pl.BlockSpec((BM, BN), lambda m, n: (m, n)),
        pl.BlockSpec((1, BM, 1), lambda m, n: (m, 0, 0)),
        pl.BlockSpec((N_EMB, EMB_DIM), lambda m, n: (0, 0)),
    ],
    out_shape=[
        jax.ShapeDtypeStruct((N_TOK, N_EMB), jnp.float32),
        jax.ShapeDtypeStruct((MB_A, BM, 1), jnp.int32),
        jax.ShapeDtypeStruct((N_EMB, EMB_DIM), jnp.float32),
    ],
    scratch_shapes=[
        pltpu.VMEM((BM, 1), jnp.float32),
        pltpu.VMEM((BM, 1), jnp.int32),
    ],
    compiler_params=pltpu.CompilerParams(
        dimension_semantics=("arbitrary", "arbitrary")),
)

BM_C, BN_C = 2048, 2048
MB_C, NB_C = N_TOK // BM_C, N_EMB // BN_C


def _onehot_body(idx_ref, enc_ref, perp_ref, colsum, ent):
    nn = pl.program_id(0)
    mm = pl.program_id(1)
    idx = idx_ref[0]
    cols = nn * BN_C + lax.broadcasted_iota(jnp.int32, (BM_C, BN_C), 1)
    oh = (idx == cols).astype(jnp.float32)
    enc_ref[...] = oh
    cs = jnp.sum(oh, axis=0, keepdims=True)

    @pl.when(mm == 0)
    def _():
        colsum[...] = cs

    @pl.when(mm > 0)
    def _():
        colsum[...] = colsum[...] + cs

    @pl.when(mm == MB_C - 1)
    def _():
        p = colsum[...] * (1.0 / N_TOK)
        ent_part = jnp.sum(p * jnp.log(p + 1e-10), axis=1, keepdims=True)
        prev = jnp.where(nn == 0, jnp.zeros_like(ent_part), ent[...])
        ent[...] = prev + ent_part

    @pl.when((mm == MB_C - 1) & (nn == NB_C - 1))
    def _():
        perp_ref[...] = jnp.exp(-ent[...])


_onehot_call = pl.pallas_call(
    _onehot_body,
    grid=(NB_C, MB_C),
    in_specs=[
        pl.BlockSpec((1, BM_C, 1), lambda nn, mm: (mm, 0, 0)),
    ],
    out_specs=[
        pl.BlockSpec((BM_C, BN_C), lambda nn, mm: (mm, nn)),
        pl.BlockSpec((1, 1), lambda nn, mm: (0, 0)),
    ],
    out_shape=[
        jax.ShapeDtypeStruct((N_TOK, N_EMB), jnp.float32),
        jax.ShapeDtypeStruct((1, 1), jnp.float32),
    ],
    scratch_shapes=[
        pltpu.VMEM((1, BN_C), jnp.float32),
        pltpu.VMEM((1, 1), jnp.float32),
    ],
    compiler_params=pltpu.CompilerParams(
        dimension_semantics=("arbitrary", "arbitrary")),
)

BM_D = 1024
MB_D = N_TOK // BM_D


def _loss_body(x_ref, q_ref, loss_ref, acc):
    i = pl.program_id(0)
    d = x_ref[...] - q_ref[...]
    s = jnp.sum(d * d).reshape(1, 1)
    prev = jnp.where(i == 0, jnp.zeros_like(s), acc[...])
    tot = prev + s
    acc[...] = tot

    @pl.when(i == MB_D - 1)
    def _():
        loss_ref[...] = tot * (1.25 / (N_TOK * EMB_DIM))


_loss_call = pl.pallas_call(
    _loss_body,
    grid=(MB_D,),
    in_specs=[
        pl.BlockSpec((BM_D, EMB_DIM), lambda i: (i, 0)),
        pl.BlockSpec((BM_D, EMB_DIM), lambda i: (i, 0)),
    ],
    out_specs=pl.BlockSpec((1, 1), lambda i: (0, 0)),
    out_shape=jax.ShapeDtypeStruct((1, 1), jnp.float32),
    scratch_shapes=[pltpu.VMEM((1, 1), jnp.float32)],
    compiler_params=pltpu.CompilerParams(
        dimension_semantics=("arbitrary",)),
)


@functools.lru_cache(maxsize=1)
def _make_sc_gather():
    info = plsc.get_sparse_core_info()
    nc, ns = info.num_cores, info.num_subcores
    nw = nc * ns
    bpw = N_TOK // nw
    mesh = plsc.VectorSubcoreMesh(core_axis_name="c", subcore_axis_name="s")

    @functools.partial(
        pl.kernel, mesh=mesh,
        out_type=jax.ShapeDtypeStruct((N_TOK, EMB_DIM), jnp.float32),
        scratch_types=[
            pltpu.VMEM((bpw,), jnp.int32),
            pltpu.VMEM((bpw, EMB_DIM), jnp.float32),
            pltpu.SemaphoreType.DMA,
        ],
    )
    def gk(table_hbm, idx_hbm, out_hbm, idx_v, rows_v, sem):
        wid = lax.axis_index("s") * nc + lax.axis_index("c")
        base = wid * bpw
        pltpu.sync_copy(idx_hbm.at[pl.ds(base, bpw)], idx_v)
        pltpu.async_copy(table_hbm.at[idx_v], rows_v, sem).wait()
        pltpu.sync_copy(rows_v, out_hbm.at[pl.ds(base, bpw)])

    return gk


def kernel(inputs, embeddings):
    flat = inputs.reshape(N_TOK, EMB_DIM)
    distances, idx3, table = _dist_call(flat, embeddings)
    idx_flat = idx3.reshape(N_TOK)
    encodings, perp = _onehot_call(idx3.reshape(MB_C, BM_C, 1))
    quantized = _make_sc_gather()(table, idx_flat)
    loss = _loss_call(flat, quantized)
    quantized_st = quantized.reshape(inputs.shape)
    encoding_indices = idx_flat.reshape(inputs.shape[:-1])
    return (quantized_st, loss.reshape(()), perp.reshape(()),
            encodings, encoding_indices, distances)


# speedup vs baseline: 1.2343x; 1.0369x over previous
import functools

import jax
import jax.numpy as jnp
from jax import lax
from jax.experimental import pallas as pl
from jax.experimental.pallas import tpu as pltpu
from jax.experimental.pallas import tpu_sc as plsc

N_TOK = 8192
EMB_DIM = 256
N_EMB = 8192

BM, BN = 2048, 2048
MB_A, NB_A = N_TOK // BM, N_EMB // BN


def _dist_body(x_ref, e_ref, dist_ref, idx_ref, et_ref, minv, argv):
    mdim = pl.program_id(0)
    n = pl.program_id(1)
    x = x_ref[...]
    e = e_ref[...]

    @pl.when(mdim == 0)
    def _():
        et_ref[pl.ds(n * BN, BN), :] = e.T

    m = lax.dot_general(x, e, (((1,), (0,)), ((), ())),
                        preferred_element_type=jnp.float32)
    x2 = jnp.sum(jnp.square(x), axis=1, keepdims=True)
    e2 = jnp.sum(jnp.square(e), axis=0, keepdims=True)
    dist = (x2 - 2.0 * m) + e2
    dist_ref[...] = dist

    lmin = jnp.min(dist, axis=1, keepdims=True)
    cols = n * BN + lax.broadcasted_iota(jnp.int32, (BM, BN), 1)
    larg = jnp.min(jnp.where(dist == lmin, cols, jnp.int32(2147483647)),
                   axis=1, keepdims=True)

    @pl.when(n == 0)
    def _():
        minv[...] = lmin
        argv[...] = larg

    @pl.when(n > 0)
    def _():
        better = lmin < minv[...]
        argv[...] = jnp.where(better, larg, argv[...])
        minv[...] = jnp.minimum(lmin, minv[...])

    @pl.when(n == NB_A - 1)
    def _():
        idx_ref[0] = argv[...]


_dist_call = pl.pallas_call(
    _dist_body,
    grid=(MB_A, NB_A),
    in_specs=[
        pl.BlockSpec((BM, EMB_DIM), lambda m, n: (m, 0)),
        pl.BlockSpec((EMB_DIM, BN), lambda m, n: (0, n)),
    ],
    out_specs=[
        pl.BlockSpec((BM, BN), lambda m, n: (m, n)),
        pl.BlockSpec((1, BM, 1), lambda m, n: (m, 0, 0)),
        pl.BlockSpec((N_EMB, EMB_DIM), lambda m, n: (0, 0)),
    ],
    out_shape=[
        jax.ShapeDtypeStruct((N_TOK, N_EMB), jnp.float32),
        jax.ShapeDtypeStruct((MB_A, BM, 1), jnp.int32),
        jax.ShapeDtypeStruct((N_EMB, EMB_DIM), jnp.float32),
    ],
    scratch_shapes=[
        pltpu.VMEM((BM, 1), jnp.float32),
        pltpu.VMEM((BM, 1), jnp.int32),
    ],
    compiler_params=pltpu.CompilerParams(
        dimension_semantics=("arbitrary", "arbitrary")),
)

BM_C, BN_C = 2048, 2048
MB_C, NB_C = N_TOK // BM_C, N_EMB // BN_C


def _onehot_body(idx_ref, enc_ref, perp_ref, colsum, ent):
    nn = pl.program_id(0)
    mm = pl.program_id(1)
    idx = idx_ref[0]
    cols = nn * BN_C + lax.broadcasted_iota(jnp.int32, (BM_C, BN_C), 1)
    oh = (idx == cols).astype(jnp.float32)
    enc_ref[...] = oh
    cs = jnp.sum(oh, axis=0, keepdims=True)

    @pl.when(mm == 0)
    def _():
        colsum[...] = cs

    @pl.when(mm > 0)
    def _():
        colsum[...] = colsum[...] + cs

    @pl.when(mm == MB_C - 1)
    def _():
        p = colsum[...] * (1.0 / N_TOK)
        ent_part = jnp.sum(p * jnp.log(p + 1e-10), axis=1, keepdims=True)
        prev = jnp.where(nn == 0, jnp.zeros_like(ent_part), ent[...])
        ent[...] = prev + ent_part

    @pl.when((mm == MB_C - 1) & (nn == NB_C - 1))
    def _():
        perp_ref[...] = jnp.exp(-ent[...])


_onehot_call = pl.pallas_call(
    _onehot_body,
    grid=(NB_C, MB_C),
    in_specs=[
        pl.BlockSpec((1, BM_C, 1), lambda nn, mm: (mm, 0, 0)),
    ],
    out_specs=[
        pl.BlockSpec((BM_C, BN_C), lambda nn, mm: (mm, nn)),
        pl.BlockSpec((1, 1), lambda nn, mm: (0, 0)),
    ],
    out_shape=[
        jax.ShapeDtypeStruct((N_TOK, N_EMB), jnp.float32),
        jax.ShapeDtypeStruct((1, 1), jnp.float32),
    ],
    scratch_shapes=[
        pltpu.VMEM((1, BN_C), jnp.float32),
        pltpu.VMEM((1, 1), jnp.float32),
    ],
    compiler_params=pltpu.CompilerParams(
        dimension_semantics=("arbitrary", "arbitrary")),
)

BM_D = 1024
MB_D = N_TOK // BM_D


def _loss_body(x_ref, q_ref, loss_ref, acc):
    i = pl.program_id(0)
    d = x_ref[...] - q_ref[...]
    s = jnp.sum(d * d).reshape(1, 1)
    prev = jnp.where(i == 0, jnp.zeros_like(s), acc[...])
    tot = prev + s
    acc[...] = tot

    @pl.when(i == MB_D - 1)
    def _():
        loss_ref[...] = tot * (1.25 / (N_TOK * EMB_DIM))


_loss_call = pl.pallas_call(
    _loss_body,
    grid=(MB_D,),
    in_specs=[
        pl.BlockSpec((BM_D, EMB_DIM), lambda i: (i, 0)),
        pl.BlockSpec((BM_D, EMB_DIM), lambda i: (i, 0)),
    ],
    out_specs=pl.BlockSpec((1, 1), lambda i: (0, 0)),
    out_shape=jax.ShapeDtypeStruct((1, 1), jnp.float32),
    scratch_shapes=[pltpu.VMEM((1, 1), jnp.float32)],
    compiler_params=pltpu.CompilerParams(
        dimension_semantics=("arbitrary",)),
)


@functools.lru_cache(maxsize=1)
def _make_sc_gather():
    info = plsc.get_sparse_core_info()
    nc, ns = info.num_cores, info.num_subcores
    nw = nc * ns
    bpw = N_TOK // nw
    mesh = plsc.VectorSubcoreMesh(core_axis_name="c", subcore_axis_name="s")

    @functools.partial(
        pl.kernel, mesh=mesh,
        out_type=jax.ShapeDtypeStruct((N_TOK, EMB_DIM), jnp.float32),
        scratch_types=[
            pltpu.VMEM((bpw,), jnp.int32),
            pltpu.VMEM((bpw, EMB_DIM), jnp.float32),
            pltpu.SemaphoreType.DMA,
        ],
    )
    def gk(table_hbm, idx_hbm, out_hbm, idx_v, rows_v, sem):
        wid = lax.axis_index("s") * nc + lax.axis_index("c")
        base = wid * bpw
        pltpu.sync_copy(idx_hbm.at[pl.ds(base, bpw)], idx_v)
        pltpu.async_copy(table_hbm.at[idx_v], rows_v, sem).wait()
        pltpu.sync_copy(rows_v, out_hbm.at[pl.ds(base, bpw)])

    return gk


def kernel(inputs, embeddings):
    flat = inputs.reshape(N_TOK, EMB_DIM)
    distances, idx3, table = _dist_call(flat, embeddings)
    idx_flat = idx3.reshape(N_TOK)
    encodings, perp = _onehot_call(idx3.reshape(MB_C, BM_C, 1))
    quantized = _make_sc_gather()(table, idx_flat)
    loss = _loss_call(flat, quantized)
    quantized_st = quantized.reshape(inputs.shape)
    encoding_indices = idx_flat.reshape(inputs.shape[:-1])
    return (quantized_st, loss.reshape(()), perp.reshape(()),
            encodings, encoding_indices, distances)


# speedup vs baseline: 1.2524x; 1.0147x over previous
import functools

import jax
import jax.numpy as jnp
from jax import lax
from jax.experimental import pallas as pl
from jax.experimental.pallas import tpu as pltpu
from jax.experimental.pallas import tpu_sc as plsc

N_TOK = 8192
EMB_DIM = 256
N_EMB = 8192

BM, BN = 2048, 2048
MB_A, NB_A = N_TOK // BM, N_EMB // BN


def _dist_body(x_ref, e_ref, dist_ref, idx_ref, et_ref, minv, argv):
    mdim = pl.program_id(0)
    n = pl.program_id(1)
    x = x_ref[...]
    e = e_ref[...]

    @pl.when(mdim == 0)
    def _():
        et_ref[pl.ds(n * BN, BN), :] = e.T

    m = lax.dot_general(x, e, (((1,), (0,)), ((), ())),
                        preferred_element_type=jnp.float32)
    x2 = jnp.sum(jnp.square(x), axis=1, keepdims=True)
    e2 = jnp.sum(jnp.square(e), axis=0, keepdims=True)
    dist = (x2 - 2.0 * m) + e2
    dist_ref[...] = dist

    lmin = jnp.min(dist, axis=1, keepdims=True)
    cols = n * BN + lax.broadcasted_iota(jnp.int32, (BM, BN), 1)
    larg = jnp.min(jnp.where(dist == lmin, cols, jnp.int32(2147483647)),
                   axis=1, keepdims=True)

    @pl.when(n == 0)
    def _():
        minv[...] = lmin
        argv[...] = larg

    @pl.when(n > 0)
    def _():
        better = lmin < minv[...]
        argv[...] = jnp.where(better, larg, argv[...])
        minv[...] = jnp.minimum(lmin, minv[...])

    @pl.when(n == NB_A - 1)
    def _():
        idx_ref[0] = argv[...]


_dist_call = pl.pallas_call(
    _dist_body,
    grid=(MB_A, NB_A),
    in_specs=[
        pl.BlockSpec((BM, EMB_DIM), lambda m, n: (m, 0)),
        pl.BlockSpec((EMB_DIM, BN), lambda m, n: (0, n)),
    ],
    out_specs=[
        pl.BlockSpec((BM, BN), lambda m, n: (m, n)),
        pl.BlockSpec((1, BM, 1), lambda m, n: (m, 0, 0)),
        pl.BlockSpec((N_EMB, EMB_DIM), lambda m, n: (0, 0)),
    ],
    out_shape=[
        jax.ShapeDtypeStruct((N_TOK, N_EMB), jnp.float32),
        jax.ShapeDtypeStruct((MB_A, BM, 1), jnp.int32),
        jax.ShapeDtypeStruct((N_EMB, EMB_DIM), jnp.float32),
    ],
    scratch_shapes=[
        pltpu.VMEM((BM, 1), jnp.float32),
        pltpu.VMEM((BM, 1), jnp.int32),
    ],
    compiler_params=pltpu.CompilerParams(
        dimension_semantics=("arbitrary", "arbitrary")),
)

BM_C, BN_C = 2048, 2048
MB_C, NB_C = N_TOK // BM_C, N_EMB // BN_C


def _onehot_body(idx_ref, enc_ref, perp_ref, colsum, ent):
    nn = pl.program_id(0)
    mm = pl.program_id(1)
    idx = idx_ref[0]
    cols = nn * BN_C + lax.broadcasted_iota(jnp.int32, (BM_C, BN_C), 1)
    oh = (idx == cols).astype(jnp.float32)
    enc_ref[...] = oh
    cs = jnp.sum(oh, axis=0, keepdims=True)

    @pl.when(mm == 0)
    def _():
        colsum[...] = cs

    @pl.when(mm > 0)
    def _():
        colsum[...] = colsum[...] + cs

    @pl.when(mm == MB_C - 1)
    def _():
        p = colsum[...] * (1.0 / N_TOK)
        ent_part = jnp.sum(p * jnp.log(p + 1e-10), axis=1, keepdims=True)
        prev = jnp.where(nn == 0, jnp.zeros_like(ent_part), ent[...])
        ent[...] = prev + ent_part

    @pl.when((mm == MB_C - 1) & (nn == NB_C - 1))
    def _():
        perp_ref[...] = jnp.exp(-ent[...])


_onehot_call = pl.pallas_call(
    _onehot_body,
    grid=(NB_C, MB_C),
    in_specs=[
        pl.BlockSpec((1, BM_C, 1), lambda nn, mm: (mm, 0, 0)),
    ],
    out_specs=[
        pl.BlockSpec((BM_C, BN_C), lambda nn, mm: (mm, nn)),
        pl.BlockSpec((1, 1), lambda nn, mm: (0, 0)),
    ],
    out_shape=[
        jax.ShapeDtypeStruct((N_TOK, N_EMB), jnp.float32),
        jax.ShapeDtypeStruct((1, 1), jnp.float32),
    ],
    scratch_shapes=[
        pltpu.VMEM((1, BN_C), jnp.float32),
        pltpu.VMEM((1, 1), jnp.float32),
    ],
    compiler_params=pltpu.CompilerParams(
        dimension_semantics=("arbitrary", "arbitrary")),
)


def _loss_fin_body(p_ref, loss_ref):
    s = jnp.sum(p_ref[...], axis=1, keepdims=True)
    tot = jnp.sum(s, axis=0, keepdims=True)
    loss_ref[...] = tot * (1.25 / (N_TOK * EMB_DIM))


_loss_fin = pl.pallas_call(
    _loss_fin_body,
    out_shape=jax.ShapeDtypeStruct((1, 1), jnp.float32),
)

_LCHUNK = 128


@functools.lru_cache(maxsize=1)
def _make_sc_gather():
    info = plsc.get_sparse_core_info()
    nc, ns = info.num_cores, info.num_subcores
    nw = nc * ns
    bpw = N_TOK // nw
    nchunk = bpw // _LCHUNK
    mesh = plsc.VectorSubcoreMesh(core_axis_name="c", subcore_axis_name="s")

    @functools.partial(
        pl.kernel, mesh=mesh,
        out_type=[
            jax.ShapeDtypeStruct((N_TOK, EMB_DIM), jnp.float32),
            jax.ShapeDtypeStruct((nw * 16,), jnp.float32),
        ],
        scratch_types=[
            pltpu.VMEM((bpw,), jnp.int32),
            pltpu.VMEM((bpw, EMB_DIM), jnp.float32),
            pltpu.VMEM((_LCHUNK, EMB_DIM), jnp.float32),
            pltpu.VMEM((16,), jnp.float32),
            pltpu.SemaphoreType.DMA,
        ],
    )
    def gk(table_hbm, idx_hbm, x_hbm, out_hbm, part_hbm,
           idx_v, rows_v, x_v, part_v, sem):
        wid = lax.axis_index("s") * nc + lax.axis_index("c")
        base = wid * bpw
        pltpu.sync_copy(idx_hbm.at[pl.ds(base, bpw)], idx_v)
        pltpu.async_copy(table_hbm.at[idx_v], rows_v, sem).wait()
        pltpu.sync_copy(rows_v, out_hbm.at[pl.ds(base, bpw)])
        acc0 = jnp.zeros((16,), jnp.float32)
        for c in range(nchunk):
            pltpu.sync_copy(x_hbm.at[pl.ds(base + c * _LCHUNK, _LCHUNK)], x_v)

            def body(i, acc, _c=c):
                for l in range(EMB_DIM // 16):
                    xv = x_v[i, pl.ds(l * 16, 16)]
                    rv = rows_v[_c * _LCHUNK + i, pl.ds(l * 16, 16)]
                    dd = rv - xv
                    acc = acc + dd * dd
                return acc

            acc0 = lax.fori_loop(0, _LCHUNK, body, acc0)
        part_v[...] = acc0
        pltpu.sync_copy(part_v, part_hbm.at[pl.ds(wid * 16, 16)])

    return gk


def kernel(inputs, embeddings):
    flat = inputs.reshape(N_TOK, EMB_DIM)
    distances, idx3, table = _dist_call(flat, embeddings)
    idx_flat = idx3.reshape(N_TOK)
    encodings, perp = _onehot_call(idx3.reshape(MB_C, BM_C, 1))
    quantized, parts = _make_sc_gather()(table, idx_flat, flat)
    loss = _loss_fin(parts.reshape(4, 128))
    quantized_st = quantized.reshape(inputs.shape)
    encoding_indices = idx_flat.reshape(inputs.shape[:-1])
    return (quantized_st, loss.reshape(()), perp.reshape(()),
            encodings, encoding_indices, distances)


# speedup vs baseline: 1.2955x; 1.0344x over previous
import functools

import jax
import jax.numpy as jnp
from jax import lax
from jax.experimental import pallas as pl
from jax.experimental.pallas import tpu as pltpu
from jax.experimental.pallas import tpu_sc as plsc

N_TOK = 8192
EMB_DIM = 256
N_EMB = 8192

BM, BN = 2048, 2048
MB_A, NB_A = N_TOK // BM, N_EMB // BN


def _dist_body(x_ref, e_ref, dist_ref, idx_ref, et_ref, minv, argv):
    mdim = pl.program_id(0)
    n = pl.program_id(1)
    x = x_ref[...]
    e = e_ref[...]

    @pl.when(mdim == 0)
    def _():
        et_ref[pl.ds(n * BN, BN), :] = e.T

    m = lax.dot_general(x, e, (((1,), (0,)), ((), ())),
                        preferred_element_type=jnp.float32)
    x2 = jnp.sum(jnp.square(x), axis=1, keepdims=True)
    e2 = jnp.sum(jnp.square(e), axis=0, keepdims=True)
    dist = (x2 - 2.0 * m) + e2
    dist_ref[...] = dist

    lmin = jnp.min(dist, axis=1, keepdims=True)
    cols = n * BN + lax.broadcasted_iota(jnp.int32, (BM, BN), 1)
    larg = jnp.min(jnp.where(dist == lmin, cols, jnp.int32(2147483647)),
                   axis=1, keepdims=True)

    @pl.when(n == 0)
    def _():
        minv[...] = lmin
        argv[...] = larg

    @pl.when(n > 0)
    def _():
        better = lmin < minv[...]
        argv[...] = jnp.where(better, larg, argv[...])
        minv[...] = jnp.minimum(lmin, minv[...])

    @pl.when(n == NB_A - 1)
    def _():
        idx_ref[0] = argv[...].reshape(1, BM)


_dist_call = pl.pallas_call(
    _dist_body,
    grid=(MB_A, NB_A),
    in_specs=[
        pl.BlockSpec((BM, EMB_DIM), lambda m, n: (m, 0)),
        pl.BlockSpec((EMB_DIM, BN), lambda m, n: (0, n)),
    ],
    out_specs=[
        pl.BlockSpec((BM, BN), lambda m, n: (m, n)),
        pl.BlockSpec((1, 1, BM), lambda m, n: (m, 0, 0)),
        pl.BlockSpec((N_EMB, EMB_DIM), lambda m, n: (0, 0)),
    ],
    out_shape=[
        jax.ShapeDtypeStruct((N_TOK, N_EMB), jnp.float32),
        jax.ShapeDtypeStruct((MB_A, 1, BM), jnp.int32),
        jax.ShapeDtypeStruct((N_EMB, EMB_DIM), jnp.float32),
    ],
    scratch_shapes=[
        pltpu.VMEM((BM, 1), jnp.float32),
        pltpu.VMEM((BM, 1), jnp.int32),
    ],
    compiler_params=pltpu.CompilerParams(
        dimension_semantics=("arbitrary", "arbitrary")),
)

BM_C, BN_C = 2048, 2048
MB_C, NB_C = N_TOK // BM_C, N_EMB // BN_C


def _onehot_body(idx_ref, enc_ref, perp_ref, colsum, ent):
    nn = pl.program_id(0)
    mm = pl.program_id(1)
    idx = idx_ref[0].reshape(BM_C, 1)
    cols = nn * BN_C + lax.broadcasted_iota(jnp.int32, (BM_C, BN_C), 1)
    oh = (idx == cols).astype(jnp.float32)
    enc_ref[...] = oh
    cs = jnp.sum(oh, axis=0, keepdims=True)

    @pl.when(mm == 0)
    def _():
        colsum[...] = cs

    @pl.when(mm > 0)
    def _():
        colsum[...] = colsum[...] + cs

    @pl.when(mm == MB_C - 1)
    def _():
        p = colsum[...] * (1.0 / N_TOK)
        ent_part = jnp.sum(p * jnp.log(p + 1e-10), axis=1, keepdims=True)
        prev = jnp.where(nn == 0, jnp.zeros_like(ent_part), ent[...])
        ent[...] = prev + ent_part

    @pl.when((mm == MB_C - 1) & (nn == NB_C - 1))
    def _():
        perp_ref[...] = jnp.exp(-ent[...])


_onehot_call = pl.pallas_call(
    _onehot_body,
    grid=(NB_C, MB_C),
    in_specs=[
        pl.BlockSpec((1, 1, BM_C), lambda nn, mm: (mm, 0, 0)),
    ],
    out_specs=[
        pl.BlockSpec((BM_C, BN_C), lambda nn, mm: (mm, nn)),
        pl.BlockSpec((1, 1), lambda nn, mm: (0, 0)),
    ],
    out_shape=[
        jax.ShapeDtypeStruct((N_TOK, N_EMB), jnp.float32),
        jax.ShapeDtypeStruct((1, 1), jnp.float32),
    ],
    scratch_shapes=[
        pltpu.VMEM((1, BN_C), jnp.float32),
        pltpu.VMEM((1, 1), jnp.float32),
    ],
    compiler_params=pltpu.CompilerParams(
        dimension_semantics=("arbitrary", "arbitrary")),
)


def _loss_fin_body(p_ref, loss_ref):
    s = jnp.sum(p_ref[...], axis=1, keepdims=True)
    tot = jnp.sum(s, axis=0, keepdims=True)
    loss_ref[...] = tot * (1.25 / (N_TOK * EMB_DIM))


_loss_fin = pl.pallas_call(
    _loss_fin_body,
    out_shape=jax.ShapeDtypeStruct((1, 1), jnp.float32),
)

_LCHUNK = 128


@functools.lru_cache(maxsize=1)
def _make_sc_gather():
    info = plsc.get_sparse_core_info()
    nc, ns = info.num_cores, info.num_subcores
    nw = nc * ns
    bpw = N_TOK // nw
    nchunk = bpw // _LCHUNK
    mesh = plsc.VectorSubcoreMesh(core_axis_name="c", subcore_axis_name="s")

    @functools.partial(
        pl.kernel, mesh=mesh,
        out_type=[
            jax.ShapeDtypeStruct((N_TOK, EMB_DIM), jnp.float32),
            jax.ShapeDtypeStruct((nw * 16,), jnp.float32),
        ],
        scratch_types=[
            pltpu.VMEM((bpw,), jnp.int32),
            pltpu.VMEM((bpw, EMB_DIM), jnp.float32),
            pltpu.VMEM((_LCHUNK, EMB_DIM), jnp.float32),
            pltpu.VMEM((16,), jnp.float32),
            pltpu.SemaphoreType.DMA,
        ],
    )
    def gk(table_hbm, idx_hbm, x_hbm, out_hbm, part_hbm,
           idx_v, rows_v, x_v, part_v, sem):
        wid = lax.axis_index("s") * nc + lax.axis_index("c")
        base = wid * bpw
        pltpu.sync_copy(idx_hbm.at[pl.ds(base, bpw)], idx_v)
        pltpu.async_copy(table_hbm.at[idx_v], rows_v, sem).wait()
        pltpu.sync_copy(rows_v, out_hbm.at[pl.ds(base, bpw)])
        acc0 = jnp.zeros((16,), jnp.float32)
        for c in range(nchunk):
            pltpu.sync_copy(x_hbm.at[pl.ds(base + c * _LCHUNK, _LCHUNK)], x_v)

            def body(i, acc, _c=c):
                for l in range(EMB_DIM // 16):
                    xv = x_v[i, pl.ds(l * 16, 16)]
                    rv = rows_v[_c * _LCHUNK + i, pl.ds(l * 16, 16)]
                    dd = rv - xv
                    acc = acc + dd * dd
                return acc

            acc0 = lax.fori_loop(0, _LCHUNK, body, acc0)
        part_v[...] = acc0
        pltpu.sync_copy(part_v, part_hbm.at[pl.ds(wid * 16, 16)])

    return gk


def kernel(inputs, embeddings):
    flat = inputs.reshape(N_TOK, EMB_DIM)
    distances, idx3, table = _dist_call(flat, embeddings)
    idx_flat = idx3.reshape(N_TOK)
    encodings, perp = _onehot_call(idx3.reshape(MB_C, 1, BM_C))
    quantized, parts = _make_sc_gather()(table, idx_flat, flat)
    loss = _loss_fin(parts.reshape(4, 128))
    quantized_st = quantized.reshape(inputs.shape)
    encoding_indices = idx_flat.reshape(inputs.shape[:-1])
    return (quantized_st, loss.reshape(()), perp.reshape(()),
            encodings, encoding_indices, distances)


# speedup vs baseline: 1.3315x; 1.0278x over previous
import functools

import jax
import jax.numpy as jnp
from jax import lax
from jax.experimental import pallas as pl
from jax.experimental.pallas import tpu as pltpu
from jax.experimental.pallas import tpu_sc as plsc

N_TOK = 8192
EMB_DIM = 256
N_EMB = 8192

BM, BN = 2048, 2048
MB_A, NB_A = N_TOK // BM, N_EMB // BN


def _dist_body(x_ref, e_ref, dist_ref, idx_ref, et_ref, minv, argv):
    mdim = pl.program_id(0)
    n = pl.program_id(1)
    x = x_ref[...]
    e = e_ref[...]

    @pl.when(mdim == 0)
    def _():
        et_ref[pl.ds(n * BN, BN), :] = e.T

    m = lax.dot_general(-2.0 * x, e, (((1,), (0,)), ((), ())),
                        preferred_element_type=jnp.float32)
    x2 = jnp.sum(jnp.square(x), axis=1, keepdims=True)
    e2 = jnp.sum(jnp.square(e), axis=0, keepdims=True)
    dist = (x2 + m) + e2
    dist_ref[...] = dist

    lmin = jnp.min(dist, axis=1, keepdims=True)
    cols = lax.broadcasted_iota(jnp.int32, (BM, BN), 1)
    larg = (n * BN) + jnp.min(
        jnp.where(dist == lmin, cols, jnp.int32(2147483647)),
        axis=1, keepdims=True)

    @pl.when(n == 0)
    def _():
        minv[...] = lmin
        argv[...] = larg

    @pl.when(n > 0)
    def _():
        better = lmin < minv[...]
        argv[...] = jnp.where(better, larg, argv[...])
        minv[...] = jnp.minimum(lmin, minv[...])

    @pl.when(n == NB_A - 1)
    def _():
        idx_ref[0] = argv[...].reshape(1, BM)


_dist_call = pl.pallas_call(
    _dist_body,
    grid=(MB_A, NB_A),
    in_specs=[
        pl.BlockSpec((BM, EMB_DIM), lambda m, n: (m, 0)),
        pl.BlockSpec((EMB_DIM, BN), lambda m, n: (0, n)),
    ],
    out_specs=[
        pl.BlockSpec((BM, BN), lambda m, n: (m, n)),
        pl.BlockSpec((1, 1, BM), lambda m, n: (m, 0, 0)),
        pl.BlockSpec((N_EMB, EMB_DIM), lambda m, n: (0, 0)),
    ],
    out_shape=[
        jax.ShapeDtypeStruct((N_TOK, N_EMB), jnp.float32),
        jax.ShapeDtypeStruct((MB_A, 1, BM), jnp.int32),
        jax.ShapeDtypeStruct((N_EMB, EMB_DIM), jnp.float32),
    ],
    scratch_shapes=[
        pltpu.VMEM((BM, 1), jnp.float32),
        pltpu.VMEM((BM, 1), jnp.int32),
    ],
    compiler_params=pltpu.CompilerParams(
        dimension_semantics=("arbitrary", "arbitrary")),
)

BM_C, BN_C = 2048, 2048
MB_C, NB_C = N_TOK // BM_C, N_EMB // BN_C


def _onehot_body(idx_ref, enc_ref, perp_ref, colsum, ent):
    nn = pl.program_id(0)
    mm = pl.program_id(1)
    idx = idx_ref[0].reshape(BM_C, 1)
    cols = nn * BN_C + lax.broadcasted_iota(jnp.int32, (BM_C, BN_C), 1)
    oh = (idx == cols).astype(jnp.float32)
    enc_ref[...] = oh
    cs = jnp.sum(oh, axis=0, keepdims=True)

    @pl.when(mm == 0)
    def _():
        colsum[...] = cs

    @pl.when(mm > 0)
    def _():
        colsum[...] = colsum[...] + cs

    @pl.when(mm == MB_C - 1)
    def _():
        p = colsum[...] * (1.0 / N_TOK)
        ent_part = jnp.sum(p * jnp.log(p + 1e-10), axis=1, keepdims=True)
        prev = jnp.where(nn == 0, jnp.zeros_like(ent_part), ent[...])
        ent[...] = prev + ent_part

    @pl.when((mm == MB_C - 1) & (nn == NB_C - 1))
    def _():
        perp_ref[...] = jnp.exp(-ent[...])


_onehot_call = pl.pallas_call(
    _onehot_body,
    grid=(NB_C, MB_C),
    in_specs=[
        pl.BlockSpec((1, 1, BM_C), lambda nn, mm: (mm, 0, 0)),
    ],
    out_specs=[
        pl.BlockSpec((BM_C, BN_C), lambda nn, mm: (mm, nn)),
        pl.BlockSpec((1, 1), lambda nn, mm: (0, 0)),
    ],
    out_shape=[
        jax.ShapeDtypeStruct((N_TOK, N_EMB), jnp.float32),
        jax.ShapeDtypeStruct((1, 1), jnp.float32),
    ],
    scratch_shapes=[
        pltpu.VMEM((1, BN_C), jnp.float32),
        pltpu.VMEM((1, 1), jnp.float32),
    ],
    compiler_params=pltpu.CompilerParams(
        dimension_semantics=("arbitrary", "arbitrary")),
)


def _loss_fin_body(p_ref, loss_ref):
    s = jnp.sum(p_ref[...], axis=1, keepdims=True)
    tot = jnp.sum(s, axis=0, keepdims=True)
    loss_ref[...] = tot * (1.25 / (N_TOK * EMB_DIM))


_loss_fin = pl.pallas_call(
    _loss_fin_body,
    out_shape=jax.ShapeDtypeStruct((1, 1), jnp.float32),
)

_LCHUNK = 128


@functools.lru_cache(maxsize=1)
def _make_sc_gather():
    info = plsc.get_sparse_core_info()
    nc, ns = info.num_cores, info.num_subcores
    nw = nc * ns
    bpw = N_TOK // nw
    nchunk = bpw // _LCHUNK
    mesh = plsc.VectorSubcoreMesh(core_axis_name="c", subcore_axis_name="s")

    @functools.partial(
        pl.kernel, mesh=mesh,
        out_type=[
            jax.ShapeDtypeStruct((N_TOK, EMB_DIM), jnp.float32),
            jax.ShapeDtypeStruct((nw * 16,), jnp.float32),
        ],
        scratch_types=[
            pltpu.VMEM((bpw,), jnp.int32),
            pltpu.VMEM((bpw, EMB_DIM), jnp.float32),
            pltpu.VMEM((_LCHUNK, EMB_DIM), jnp.float32),
            pltpu.VMEM((16,), jnp.float32),
            pltpu.SemaphoreType.DMA,
        ],
    )
    def gk(table_hbm, idx_hbm, x_hbm, out_hbm, part_hbm,
           idx_v, rows_v, x_v, part_v, sem):
        wid = lax.axis_index("s") * nc + lax.axis_index("c")
        base = wid * bpw
        pltpu.sync_copy(idx_hbm.at[pl.ds(base, bpw)], idx_v)
        pltpu.async_copy(table_hbm.at[idx_v], rows_v, sem).wait()
        pltpu.sync_copy(rows_v, out_hbm.at[pl.ds(base, bpw)])
        acc0 = jnp.zeros((16,), jnp.float32)
        for c in range(nchunk):
            pltpu.sync_copy(x_hbm.at[pl.ds(base + c * _LCHUNK, _LCHUNK)], x_v)

            def body(i, acc, _c=c):
                for l in range(EMB_DIM // 16):
                    xv = x_v[i, pl.ds(l * 16, 16)]
                    rv = rows_v[_c * _LCHUNK + i, pl.ds(l * 16, 16)]
                    dd = rv - xv
                    acc = acc + dd * dd
                return acc

            acc0 = lax.fori_loop(0, _LCHUNK, body, acc0)
        part_v[...] = acc0
        pltpu.sync_copy(part_v, part_hbm.at[pl.ds(wid * 16, 16)])

    return gk


def kernel(inputs, embeddings):
    flat = inputs.reshape(N_TOK, EMB_DIM)
    distances, idx3, table = _dist_call(flat, embeddings)
    idx_flat = idx3.reshape(N_TOK)
    encodings, perp = _onehot_call(idx3.reshape(MB_C, 1, BM_C))
    quantized, parts = _make_sc_gather()(table, idx_flat, flat)
    loss = _loss_fin(parts.reshape(4, 128))
    quantized_st = quantized.reshape(inputs.shape)
    encoding_indices = idx_flat.reshape(inputs.shape[:-1])
    return (quantized_st, loss.reshape(()), perp.reshape(()),
            encodings, encoding_indices, distances)
